# baseline (speedup 1.0000x reference)
import functools

import numpy as np
import jax
import jax.numpy as jnp
from jax import lax
from jax.experimental import pallas as pl
from jax.experimental.pallas import tpu as pltpu
from jax.experimental.pallas import tpu_sc as plsc

_FIELD_DIMS = [100000] * 26
_OFFSETS = np.concatenate(([0], np.cumsum(_FIELD_DIMS)[:-1])).astype(np.int32)
_F = len(_FIELD_DIMS)
_D = 16
_B = 4096
_N = _B * _F
_ROWS, _COLS = np.triu_indices(_F, k=1)

_NC, _NS = 2, 16
_NW = _NC * _NS
_BPW = _N // _NW

_SC_PARAMS = pltpu.CompilerParams(
    use_tc_tiling_on_sc=False, needs_layout_passes=False)
_RING = 8


def _sc_gather_emb(table3, idx_flat):
    mesh = plsc.VectorSubcoreMesh(core_axis_name="c", subcore_axis_name="s")

    @functools.partial(
        pl.kernel,
        mesh=mesh,
        compiler_params=pltpu.CompilerParams(
            use_tc_tiling_on_sc=True, needs_layout_passes=False),
        out_type=jax.ShapeDtypeStruct((_N * _D,), jnp.float32),
        scratch_types=(
            [pltpu.VMEM((_BPW + 16,), jnp.int32)]
            + [pltpu.VMEM((2, 8, 128), jnp.float32) for _ in range(_RING)]
            + [pltpu.VMEM((_BPW * _D,), jnp.float32)]
            + [pltpu.SemaphoreType.DMA for _ in range(_RING)]
        ),
    )
    def gather_kernel(tab_hbm, idx_hbm, out_emb, idx_v, *rest):
        bufs = rest[:_RING]
        rows_v = rest[_RING]
        sems = rest[_RING + 1:]
        wid = lax.axis_index("s") * _NC + lax.axis_index("c")
        base = wid * _BPW
        pltpu.sync_copy(idx_hbm.at[pl.ds(base, _BPW)],
                        idx_v.at[pl.ds(0, _BPW)])

        def sidx(j):
            return idx_v[pl.ds(j, 16)][0]

        def issue(j, slot):
            ct = lax.shift_right_logical(sidx(j), 7) * 128
            ct = pl.multiple_of(ct, 128)
            pltpu.async_copy(tab_hbm.at[:, :, pl.ds(ct, 128)],
                             bufs[slot], sems[slot])

        def drain(slot):
            pltpu.make_async_copy(
                tab_hbm.at[:, :, pl.ds(0, 128)],
                bufs[slot], sems[slot]).wait()

        def pick(j, slot):
            lane = lax.bitwise_and(sidx(j), 127)
            i16 = lax.iota(jnp.int32, 16)
            vals = plsc.load_gather(
                bufs[slot], [lax.shift_right_logical(i16, 3),
                             lax.bitwise_and(i16, 7),
                             jnp.full((16,), lane, jnp.int32)])
            rows_v[pl.ds(j * _D, _D)] = vals

        for r in range(_RING):
            issue(r, r)

        @pl.loop(0, _BPW, step=_RING)
        def _(j):
            for r in range(_RING):
                drain(r)
                pick(j + r, r)

                @pl.when(j + r + _RING < _BPW)
                def _():
                    issue(j + r + _RING, r)

        pltpu.sync_copy(rows_v, out_emb.at[pl.ds(base * _D, _BPW * _D)])

    return gather_kernel(table3, idx_flat)


def _sc_gather_lr(lr_view, idx_flat):
    mesh = plsc.VectorSubcoreMesh(core_axis_name="c", subcore_axis_name="s")

    @functools.partial(
        pl.kernel,
        mesh=mesh,
        compiler_params=_SC_PARAMS,
        out_type=jax.ShapeDtypeStruct((_N,), jnp.float32),
        scratch_types=[
            pltpu.VMEM((_BPW,), jnp.int32),
            pltpu.VMEM((_BPW,), jnp.int32),
            pltpu.VMEM((_BPW, _D), jnp.float32),
            pltpu.VMEM((_BPW,), jnp.float32),
            pltpu.SemaphoreType.DMA,
        ],
    )
    def gather_kernel(lrv_hbm, idx_hbm, out_lr,
                      idx_v, idx16_v, lrg_v, lrsel_v, sem):
        wid = lax.axis_index("s") * _NC + lax.axis_index("c")
        base = wid * _BPW
        pltpu.sync_copy(idx_hbm.at[pl.ds(base, _BPW)], idx_v)

        @pl.loop(0, _BPW, step=16)
        def _(k):
            idx16_v[pl.ds(k, 16)] = lax.shift_right_logical(
                idx_v[pl.ds(k, 16)], 4)

        pltpu.async_copy(lrv_hbm.at[idx16_v], lrg_v, sem).wait()

        @pl.loop(0, _BPW, step=16)
        def _(k):
            lanes = lax.bitwise_and(idx_v[pl.ds(k, 16)], 15)
            rows16 = lax.iota(jnp.int32, 16) + k
            lrsel_v[pl.ds(k, 16)] = plsc.load_gather(lrg_v, [rows16, lanes])

        pltpu.sync_copy(lrsel_v, out_lr.at[pl.ds(base, _BPW)])

    return gather_kernel(lr_view, idx_flat)


def _tc_interact(emb_flat, lr_g, w, bias2d):
    bb = 512

    def body(emb_ref, lr_ref, w_ref, b_ref, out_ref):
        e = emb_ref[...]
        acc = jnp.dot(e.astype(jnp.bfloat16), w_ref[...],
                      preferred_element_type=jnp.float32)
        fw = jnp.sum(e * acc, axis=1, keepdims=True)
        lrs = jnp.sum(lr_ref[...], axis=1, keepdims=True)
        out_ref[...] = fw + lrs + b_ref[...]

    return pl.pallas_call(
        body,
        grid=(_B // bb,),
        in_specs=[
            pl.BlockSpec((bb, _F * _D), lambda i: (i, 0)),
            pl.BlockSpec((bb, _F), lambda i: (i, 0)),
            pl.BlockSpec((_F * _D, _F * _D), lambda i: (0, 0)),
            pl.BlockSpec((1, 1), lambda i: (0, 0)),
        ],
        out_specs=pl.BlockSpec((bb, 1), lambda i: (i, 0)),
        out_shape=jax.ShapeDtypeStruct((_B, 1), jnp.float32),
    )(emb_flat, lr_g, w, bias2d)


def kernel(x, table_lr, bias, table_emb, r):
    idx = (x + jnp.asarray(_OFFSETS)[None, :]).reshape(-1)
    table3 = jnp.swapaxes(table_emb, 0, 1).reshape(2, 8, table_emb.shape[0])
    lr_view = table_lr.reshape(-1, _D)
    emb_flat1d = _sc_gather_emb(table3, idx)
    idx_lr, _ = lax.optimization_barrier((idx, emb_flat1d[:8]))
    lr_rows = _sc_gather_lr(lr_view, idx_lr)
    emb_flat = emb_flat1d.reshape(_B, _F * _D)
    lr_g = lr_rows.reshape(_B, _F)
    m = jnp.zeros((_F, _F), jnp.float32).at[_ROWS, _COLS].set(r[:, 0])
    w = jnp.kron(m, jnp.eye(_D, dtype=jnp.float32)).astype(jnp.bfloat16)
    return _tc_interact(emb_flat, lr_g, w, bias.reshape(1, 1))


# speedup vs baseline: 1.2394x; 1.2394x over previous
import functools

import numpy as np
import jax
import jax.numpy as jnp
from jax import lax
from jax.experimental import pallas as pl
from jax.experimental.pallas import tpu as pltpu
from jax.experimental.pallas import tpu_sc as plsc

_FIELD_DIMS = [100000] * 26
_OFFSETS = np.concatenate(([0], np.cumsum(_FIELD_DIMS)[:-1])).astype(np.int32)
_F = len(_FIELD_DIMS)
_D = 16
_B = 4096
_N = _B * _F
_ROWS, _COLS = np.triu_indices(_F, k=1)

_NC, _NS = 2, 16
_NW = _NC * _NS
_BPW = _N // _NW

_SC_PARAMS = pltpu.CompilerParams(
    use_tc_tiling_on_sc=False, needs_layout_passes=False)
_RING = 8


def _sc_gather_emb(table3, lr_t, idx_flat):
    mesh = plsc.VectorSubcoreMesh(core_axis_name="c", subcore_axis_name="s")

    @functools.partial(
        pl.kernel,
        mesh=mesh,
        compiler_params=pltpu.CompilerParams(
            use_tc_tiling_on_sc=True, needs_layout_passes=False),
        out_type=(
            jax.ShapeDtypeStruct((_N * _D,), jnp.float32),
            jax.ShapeDtypeStruct((_N,), jnp.float32),
        ),
        scratch_types=(
            [pltpu.VMEM((_BPW + 16,), jnp.int32)]
            + [pltpu.VMEM((2, 8, 128), jnp.float32) for _ in range(_RING)]
            + [pltpu.VMEM((1, 128), jnp.float32) for _ in range(_RING)]
            + [pltpu.VMEM((_BPW * _D,), jnp.float32),
               pltpu.VMEM((_BPW + 16,), jnp.float32)]
            + [pltpu.SemaphoreType.DMA for _ in range(_RING)]
        ),
    )
    def gather_kernel(tab_hbm, lrt_hbm, idx_hbm, out_emb, out_lr,
                      idx_v, *rest):
        bufs = rest[:_RING]
        lrbufs = rest[_RING:2 * _RING]
        rows_v = rest[2 * _RING]
        lrsel_v = rest[2 * _RING + 1]
        sems = rest[2 * _RING + 2:]
        wid = lax.axis_index("s") * _NC + lax.axis_index("c")
        base = wid * _BPW
        pltpu.sync_copy(idx_hbm.at[pl.ds(base, _BPW)],
                        idx_v.at[pl.ds(0, _BPW)])

        def sidx(j):
            return idx_v[pl.ds(j, 16)][0]

        def issue(j, slot):
            ct = lax.shift_right_logical(sidx(j), 7) * 128
            ct = pl.multiple_of(ct, 128)
            pltpu.async_copy(tab_hbm.at[:, :, pl.ds(ct, 128)],
                             bufs[slot], sems[slot])
            pltpu.async_copy(lrt_hbm.at[:, pl.ds(ct, 128)],
                             lrbufs[slot], sems[slot])

        def drain(slot):
            pltpu.make_async_copy(
                tab_hbm.at[:, :, pl.ds(0, 128)],
                bufs[slot], sems[slot]).wait()
            pltpu.make_async_copy(
                lrt_hbm.at[:, pl.ds(0, 128)],
                lrbufs[slot], sems[slot]).wait()

        def pick(j, slot):
            lane = lax.bitwise_and(sidx(j), 127)
            lane16 = jnp.full((16,), lane, jnp.int32)
            i16 = lax.iota(jnp.int32, 16)
            vals = plsc.load_gather(
                bufs[slot], [lax.shift_right_logical(i16, 3),
                             lax.bitwise_and(i16, 7), lane16])
            rows_v[pl.ds(j * _D, _D)] = vals
            lrv = plsc.load_gather(
                lrbufs[slot], [jnp.zeros((16,), jnp.int32), lane16])
            lrsel_v[pl.ds(j, 16)] = lrv

        for r in range(_RING):
            issue(r, r)

        @pl.loop(0, _BPW, step=_RING)
        def _(j):
            for r in range(_RING):
                drain(r)
                pick(j + r, r)

                @pl.when(j + r + _RING < _BPW)
                def _():
                    issue(j + r + _RING, r)

        pltpu.sync_copy(rows_v, out_emb.at[pl.ds(base * _D, _BPW * _D)])
        pltpu.sync_copy(lrsel_v.at[pl.ds(0, _BPW)],
                        out_lr.at[pl.ds(base, _BPW)])

    return gather_kernel(table3, lr_t, idx_flat)


def _sc_gather_lr(lr_view, idx_flat):
    mesh = plsc.VectorSubcoreMesh(core_axis_name="c", subcore_axis_name="s")

    @functools.partial(
        pl.kernel,
        mesh=mesh,
        compiler_params=_SC_PARAMS,
        out_type=jax.ShapeDtypeStruct((_N,), jnp.float32),
        scratch_types=[
            pltpu.VMEM((_BPW,), jnp.int32),
            pltpu.VMEM((_BPW,), jnp.int32),
            pltpu.VMEM((_BPW, _D), jnp.float32),
            pltpu.VMEM((_BPW,), jnp.float32),
            pltpu.SemaphoreType.DMA,
        ],
    )
    def gather_kernel(lrv_hbm, idx_hbm, out_lr,
                      idx_v, idx16_v, lrg_v, lrsel_v, sem):
        wid = lax.axis_index("s") * _NC + lax.axis_index("c")
        base = wid * _BPW
        pltpu.sync_copy(idx_hbm.at[pl.ds(base, _BPW)], idx_v)

        @pl.loop(0, _BPW, step=16)
        def _(k):
            idx16_v[pl.ds(k, 16)] = lax.shift_right_logical(
                idx_v[pl.ds(k, 16)], 4)

        pltpu.async_copy(lrv_hbm.at[idx16_v], lrg_v, sem).wait()

        @pl.loop(0, _BPW, step=16)
        def _(k):
            lanes = lax.bitwise_and(idx_v[pl.ds(k, 16)], 15)
            rows16 = lax.iota(jnp.int32, 16) + k
            lrsel_v[pl.ds(k, 16)] = plsc.load_gather(lrg_v, [rows16, lanes])

        pltpu.sync_copy(lrsel_v, out_lr.at[pl.ds(base, _BPW)])

    return gather_kernel(lr_view, idx_flat)


def _tc_interact(emb_flat, lr_g, w, bias2d):
    bb = 512

    def body(emb_ref, lr_ref, w_ref, b_ref, out_ref):
        e = emb_ref[...]
        acc = jnp.dot(e.astype(jnp.bfloat16), w_ref[...],
                      preferred_element_type=jnp.float32)
        fw = jnp.sum(e * acc, axis=1, keepdims=True)
        lrs = jnp.sum(lr_ref[...], axis=1, keepdims=True)
        out_ref[...] = fw + lrs + b_ref[...]

    return pl.pallas_call(
        body,
        grid=(_B // bb,),
        in_specs=[
            pl.BlockSpec((bb, _F * _D), lambda i: (i, 0)),
            pl.BlockSpec((bb, _F), lambda i: (i, 0)),
            pl.BlockSpec((_F * _D, _F * _D), lambda i: (0, 0)),
            pl.BlockSpec((1, 1), lambda i: (0, 0)),
        ],
        out_specs=pl.BlockSpec((bb, 1), lambda i: (i, 0)),
        out_shape=jax.ShapeDtypeStruct((_B, 1), jnp.float32),
    )(emb_flat, lr_g, w, bias2d)


def kernel(x, table_lr, bias, table_emb, r):
    idx = (x + jnp.asarray(_OFFSETS)[None, :]).reshape(-1)
    table3 = jnp.swapaxes(table_emb, 0, 1).reshape(2, 8, table_emb.shape[0])
    lr_t = jnp.swapaxes(table_lr, 0, 1)
    emb_flat1d, lr_rows = _sc_gather_emb(table3, lr_t, idx)
    emb_flat = emb_flat1d.reshape(_B, _F * _D)
    lr_g = lr_rows.reshape(_B, _F)
    m = jnp.zeros((_F, _F), jnp.float32).at[_ROWS, _COLS].set(r[:, 0])
    w = jnp.kron(m, jnp.eye(_D, dtype=jnp.float32)).astype(jnp.bfloat16)
    return _tc_interact(emb_flat, lr_g, w, bias.reshape(1, 1))
